# baseline (speedup 1.0000x reference)
import functools

import jax
import jax.numpy as jnp
from jax import lax
from jax.experimental import pallas as pl
from jax.experimental.pallas import tpu as pltpu
from jax.experimental.pallas import tpu_sc as plsc

_N = 10000
_E = 320000
_D = 128
_H = 16
_C = 3

_NC = 2
_NS = 16
_NT = _NC * _NS
_B = 128
_NB = 79
_EP = _NT * _NB * _B
_NPAD = 10112
_RPT = _NPAD // _NS

_BLK = 1000
_GRID = _N // _BLK


def _deg_body(dst_hbm, ones_hbm, zero_hbm, out_hbm, dst_v, ones_v, acc_sh, sem):
  cid = lax.axis_index("c")
  sid = lax.axis_index("s")
  tile = cid * _NS + sid
  pltpu.sync_copy(dst_hbm.at[tile], dst_v)
  pltpu.sync_copy(ones_hbm, ones_v)
  pltpu.sync_copy(zero_hbm.at[pl.ds(sid * _RPT, _RPT)],
                  acc_sh.at[pl.ds(sid * _RPT, _RPT)])
  plsc.subcore_barrier()

  def body(j, carry):
    pltpu.sync_copy(ones_v, acc_sh.at[dst_v.at[j]], add=True)
    return carry

  lax.fori_loop(0, _NB, body, 0)
  plsc.subcore_barrier()
  pltpu.sync_copy(acc_sh.at[pl.ds(sid * _RPT, _RPT)],
                  out_hbm.at[cid, pl.ds(sid * _RPT, _RPT)])


@functools.cache
def _deg_kernel():
  return functools.partial(
      pl.kernel,
      out_type=jax.ShapeDtypeStruct((_NC, _NPAD, 8), jnp.float32),
      mesh=plsc.VectorSubcoreMesh(core_axis_name="c", subcore_axis_name="s"),
      scratch_types=[
          pltpu.VMEM((_NB, _B), jnp.int32),
          pltpu.VMEM((_B, 8), jnp.float32),
          pltpu.VMEM_SHARED((_NPAD, 8), jnp.float32),
          pltpu.SemaphoreType.DMA,
      ],
      compiler_params=pltpu.CompilerParams(use_tc_tiling_on_sc=False),
  )(_deg_body)


_NBUF = 4


def _agg_body(hp_hbm, src_hbm, dst_hbm, zero_hbm, out_hbm,
              src_v, dst_v, rows_v, acc_sh, sem):
  cid = lax.axis_index("c")
  sid = lax.axis_index("s")
  tile = cid * _NS + sid
  pltpu.sync_copy(src_hbm.at[tile], src_v)
  pltpu.sync_copy(dst_hbm.at[tile], dst_v)
  pltpu.sync_copy(zero_hbm.at[pl.ds(sid * _RPT, _RPT)],
                  acc_sh.at[pl.ds(sid * _RPT, _RPT)])
  plsc.subcore_barrier()

  for b in range(_NBUF):
    pltpu.async_copy(hp_hbm.at[src_v.at[b]], rows_v.at[b], sem)

  def body(j, carry):
    b = lax.rem(j, _NBUF)
    pltpu.make_async_copy(hp_hbm.at[src_v.at[j]], rows_v.at[b], sem).wait()
    pltpu.sync_copy(rows_v.at[b], acc_sh.at[dst_v.at[j]], add=True)

    @pl.when(j < _NB - _NBUF)
    def _():
      pltpu.async_copy(hp_hbm.at[src_v.at[j + _NBUF]], rows_v.at[b], sem)

    return carry

  lax.fori_loop(0, _NB, body, 0)
  plsc.subcore_barrier()
  pltpu.sync_copy(acc_sh.at[pl.ds(sid * _RPT, _RPT)],
                  out_hbm.at[cid, pl.ds(sid * _RPT, _RPT)])


@functools.cache
def _agg_kernel(width):
  return functools.partial(
      pl.kernel,
      out_type=jax.ShapeDtypeStruct((_NC, _NPAD, width), jnp.float32),
      mesh=plsc.VectorSubcoreMesh(core_axis_name="c", subcore_axis_name="s"),
      scratch_types=[
          pltpu.VMEM((_NB, _B), jnp.int32),
          pltpu.VMEM((_NB, _B), jnp.int32),
          pltpu.VMEM((_NBUF, _B, width), jnp.float32),
          pltpu.VMEM_SHARED((_NPAD, width), jnp.float32),
          pltpu.SemaphoreType.DMA,
      ],
      compiler_params=pltpu.CompilerParams(use_tc_tiling_on_sc=False),
  )(_agg_body)


def _mm1_body(x_ref, w_ref, o_ref):
  o_ref[...] = jnp.dot(x_ref[...], w_ref[...],
                       preferred_element_type=jnp.float32)


def _scale_body(h_ref, d0_ref, d1_ref, hp_ref, disb_ref):
  deg = 1.0 + d0_ref[...][:, :1] + d1_ref[...][:, :1]
  dis = lax.rsqrt(deg)
  hp_ref[...] = h_ref[...] * dis
  disb_ref[...] = jnp.broadcast_to(dis, (_BLK, _H))


def _layer1_body(a0_ref, a1_ref, hp_ref, disb_ref, b1_ref, w2_ref, o_ref):
  disb = disb_ref[...]
  out1 = disb * (a0_ref[...] + a1_ref[...] + hp_ref[...]) + b1_ref[...]
  g = disb * jnp.maximum(out1, 0.0)
  o_ref[...] = jnp.dot(g, w2_ref[...], preferred_element_type=jnp.float32)


def _layer2_body(c0_ref, c1_ref, zp_ref, d8_ref, b2_ref, o_ref):
  out2 = d8_ref[...] * (c0_ref[...] + c1_ref[...] + zp_ref[...]) + b2_ref[...]
  mask = lax.broadcasted_iota(jnp.int32, (_BLK, 8), 1) < _C
  neg = jnp.float32(-1e30)
  masked = jnp.where(mask, out2, neg)
  m = jnp.max(masked, axis=1, keepdims=True)
  e = jnp.where(mask, jnp.exp(masked - m), 0.0)
  s = jnp.log(jnp.sum(e, axis=1, keepdims=True))
  o_ref[...] = out2 - m - s


def _row_spec(width):
  return pl.BlockSpec((_BLK, width), lambda i: (i, 0))


def _full_spec(shape):
  return pl.BlockSpec(shape, lambda i: tuple(0 for _ in shape))


_mm1 = pl.pallas_call(
    _mm1_body,
    grid=(_GRID,),
    in_specs=[_row_spec(_D), _full_spec((_D, _H))],
    out_specs=_row_spec(_H),
    out_shape=jax.ShapeDtypeStruct((_N, _H), jnp.float32),
)

_scale = pl.pallas_call(
    _scale_body,
    grid=(_GRID,),
    in_specs=[_row_spec(_H), _row_spec(8), _row_spec(8)],
    out_specs=[_row_spec(_H), _row_spec(_H)],
    out_shape=[jax.ShapeDtypeStruct((_N, _H), jnp.float32),
               jax.ShapeDtypeStruct((_N, _H), jnp.float32)],
)

_layer1 = pl.pallas_call(
    _layer1_body,
    grid=(_GRID,),
    in_specs=[_row_spec(_H)] * 4 + [_full_spec((1, _H)), _full_spec((_H, 8))],
    out_specs=_row_spec(8),
    out_shape=jax.ShapeDtypeStruct((_N, 8), jnp.float32),
)

_layer2 = pl.pallas_call(
    _layer2_body,
    grid=(_GRID,),
    in_specs=[_row_spec(8)] * 4 + [_full_spec((1, 8))],
    out_specs=_row_spec(8),
    out_shape=jax.ShapeDtypeStruct((_N, 8), jnp.float32),
)


@jax.jit
def kernel(x, edge_index, W1, b1, W2, b2):
  src = edge_index[0]
  dst = edge_index[1]
  pad = _EP - _E
  src_p = jnp.concatenate(
      [src, jnp.zeros((pad,), jnp.int32)]).reshape(_NT, _NB, _B)
  dst_p = jnp.concatenate(
      [dst, jnp.full((pad,), _N, jnp.int32)]).reshape(_NT, _NB, _B)

  ones8 = jnp.ones((_B, 8), jnp.float32)
  zero8 = jnp.zeros((_NPAD, 8), jnp.float32)
  zero16 = jnp.zeros((_NPAD, _H), jnp.float32)
  w2p = jnp.concatenate([W2, jnp.zeros((_H, 8 - _C), jnp.float32)], axis=1)
  b2p = jnp.concatenate([b2, jnp.zeros((8 - _C,), jnp.float32)]).reshape(1, 8)

  degp = _deg_kernel()(dst_p, ones8, zero8)
  h1 = _mm1(x, W1)
  h1p, disb = _scale(h1, degp[0, :_N], degp[1, :_N])
  agg1 = _agg_kernel(_H)(h1p, src_p, dst_p, zero16)
  zp = _layer1(agg1[0, :_N], agg1[1, :_N], h1p, disb, b1.reshape(1, _H), w2p)
  agg2 = _agg_kernel(8)(zp, src_p, dst_p, zero8)
  out = _layer2(agg2[0, :_N], agg2[1, :_N], zp, disb[:, :8], b2p)
  return out[:, :_C]


# speedup vs baseline: 1.0364x; 1.0364x over previous
import functools

import jax
import jax.numpy as jnp
from jax import lax
from jax.experimental import pallas as pl
from jax.experimental.pallas import tpu as pltpu
from jax.experimental.pallas import tpu_sc as plsc

_N = 10000
_E = 320000
_D = 128
_H = 16
_C = 3

_NC = 2
_NS = 16
_NT = _NC * _NS
_B = 128
_NB = 80
_EP = _NT * _NB * _B
_NPAD = 10112
_RPT = _NPAD // _NS
_NBUF = 8

_BLK = 1000
_GRID = _N // _BLK


def _deg_body(dst_hbm, ones_hbm, zero_hbm, out_hbm, dst_v, ones_v, acc_sh, sem):
  cid = lax.axis_index("c")
  sid = lax.axis_index("s")
  tile = cid * _NS + sid
  pltpu.sync_copy(dst_hbm.at[tile], dst_v)
  pltpu.sync_copy(ones_hbm, ones_v)
  pltpu.sync_copy(zero_hbm.at[pl.ds(sid * _RPT, _RPT)],
                  acc_sh.at[pl.ds(sid * _RPT, _RPT)])
  plsc.subcore_barrier()

  def body(j, carry):
    pltpu.sync_copy(ones_v, acc_sh.at[dst_v.at[j]], add=True)
    return carry

  lax.fori_loop(0, _NB, body, 0)
  plsc.subcore_barrier()
  pltpu.sync_copy(acc_sh.at[pl.ds(sid * _RPT, _RPT)],
                  out_hbm.at[cid, pl.ds(sid * _RPT, _RPT)])


@functools.cache
def _deg_kernel():
  return functools.partial(
      pl.kernel,
      out_type=jax.ShapeDtypeStruct((_NC, _NPAD, 8), jnp.float32),
      mesh=plsc.VectorSubcoreMesh(core_axis_name="c", subcore_axis_name="s"),
      scratch_types=[
          pltpu.VMEM((_NB, _B), jnp.int32),
          pltpu.VMEM((_B, 8), jnp.float32),
          pltpu.VMEM_SHARED((_NPAD, 8), jnp.float32),
          pltpu.SemaphoreType.DMA,
      ],
      compiler_params=pltpu.CompilerParams(use_tc_tiling_on_sc=False),
  )(_deg_body)


def _agg_body(hp_hbm, src_hbm, dst_hbm, zero_hbm, out_hbm,
              src_v, dst_v, rows_v, acc_sh, sem):
  cid = lax.axis_index("c")
  sid = lax.axis_index("s")
  tile = cid * _NS + sid
  pltpu.sync_copy(src_hbm.at[tile], src_v)
  pltpu.sync_copy(dst_hbm.at[tile], dst_v)
  pltpu.sync_copy(zero_hbm.at[pl.ds(sid * _RPT, _RPT)],
                  acc_sh.at[pl.ds(sid * _RPT, _RPT)])
  plsc.subcore_barrier()

  for b in range(_NBUF):
    pltpu.async_copy(hp_hbm.at[src_v.at[b]], rows_v.at[b], sem)

  def body(j, carry):
    b = lax.rem(j, _NBUF)
    pltpu.make_async_copy(hp_hbm.at[src_v.at[j]], rows_v.at[b], sem).wait()
    pltpu.sync_copy(rows_v.at[b], acc_sh.at[dst_v.at[j]], add=True)

    @pl.when(j < _NB - _NBUF)
    def _():
      pltpu.async_copy(hp_hbm.at[src_v.at[j + _NBUF]], rows_v.at[b], sem)

    return carry

  lax.fori_loop(0, _NB, body, 0)
  plsc.subcore_barrier()
  pltpu.sync_copy(acc_sh.at[pl.ds(sid * _RPT, _RPT)],
                  out_hbm.at[cid, pl.ds(sid * _RPT, _RPT)])


@functools.cache
def _agg_kernel(width):
  return functools.partial(
      pl.kernel,
      out_type=jax.ShapeDtypeStruct((_NC, _NPAD, width), jnp.float32),
      mesh=plsc.VectorSubcoreMesh(core_axis_name="c", subcore_axis_name="s"),
      scratch_types=[
          pltpu.VMEM((_NB, _B), jnp.int32),
          pltpu.VMEM((_NB, _B), jnp.int32),
          pltpu.VMEM((_NBUF, _B, width), jnp.float32),
          pltpu.VMEM_SHARED((_NPAD, width), jnp.float32),
          pltpu.SemaphoreType.DMA,
      ],
      compiler_params=pltpu.CompilerParams(use_tc_tiling_on_sc=False),
  )(_agg_body)


def _dis_of(d_ref):
  deg = 1.0 + d_ref[0, :, :1] + d_ref[1, :, :1]
  return lax.rsqrt(deg)


def _fwd1_body(x_ref, w_ref, d_ref, o_ref):
  h1 = jnp.dot(x_ref[...], w_ref[...], preferred_element_type=jnp.float32)
  o_ref[...] = h1 * _dis_of(d_ref)


def _layer1_body(a_ref, hp_ref, d_ref, b1_ref, w2_ref, o_ref):
  dis = _dis_of(d_ref)
  out1 = dis * (a_ref[0] + a_ref[1] + hp_ref[...]) + b1_ref[...]
  g = dis * jnp.maximum(out1, 0.0)
  o_ref[...] = jnp.dot(g, w2_ref[...], preferred_element_type=jnp.float32)


def _layer2_body(c_ref, zp_ref, d_ref, b2_ref, o_ref):
  out2 = _dis_of(d_ref) * (c_ref[0] + c_ref[1] + zp_ref[...]) + b2_ref[...]
  mask = lax.broadcasted_iota(jnp.int32, (_BLK, 8), 1) < _C
  neg = jnp.float32(-1e30)
  masked = jnp.where(mask, out2, neg)
  m = jnp.max(masked, axis=1, keepdims=True)
  e = jnp.where(mask, jnp.exp(masked - m), 0.0)
  s = jnp.log(jnp.sum(e, axis=1, keepdims=True))
  o_ref[...] = out2 - m - s


def _row_spec(width):
  return pl.BlockSpec((_BLK, width), lambda i: (i, 0))


def _part_spec(width):
  return pl.BlockSpec((_NC, _BLK, width), lambda i: (0, i, 0))


def _full_spec(shape):
  return pl.BlockSpec(shape, lambda i: tuple(0 for _ in shape))


_fwd1 = pl.pallas_call(
    _fwd1_body,
    grid=(_GRID,),
    in_specs=[_row_spec(_D), _full_spec((_D, _H)), _part_spec(8)],
    out_specs=_row_spec(_H),
    out_shape=jax.ShapeDtypeStruct((_N, _H), jnp.float32),
)

_layer1 = pl.pallas_call(
    _layer1_body,
    grid=(_GRID,),
    in_specs=[_part_spec(_H), _row_spec(_H), _part_spec(8),
              _full_spec((1, _H)), _full_spec((_H, 8))],
    out_specs=_row_spec(8),
    out_shape=jax.ShapeDtypeStruct((_N, 8), jnp.float32),
)

_layer2 = pl.pallas_call(
    _layer2_body,
    grid=(_GRID,),
    in_specs=[_part_spec(8), _row_spec(8), _part_spec(8), _full_spec((1, 8))],
    out_specs=_row_spec(8),
    out_shape=jax.ShapeDtypeStruct((_N, 8), jnp.float32),
)


@jax.jit
def kernel(x, edge_index, W1, b1, W2, b2):
  src = edge_index[0]
  dst = edge_index[1]
  pad = _EP - _E
  src_p = jnp.concatenate(
      [src, jnp.zeros((pad,), jnp.int32)]).reshape(_NT, _NB, _B)
  dst_p = jnp.concatenate(
      [dst, jnp.full((pad,), _N, jnp.int32)]).reshape(_NT, _NB, _B)

  ones8 = jnp.ones((_B, 8), jnp.float32)
  zero8 = jnp.zeros((_NPAD, 8), jnp.float32)
  zero16 = jnp.zeros((_NPAD, _H), jnp.float32)
  w2p = jnp.concatenate([W2, jnp.zeros((_H, 8 - _C), jnp.float32)], axis=1)
  b2p = jnp.concatenate([b2, jnp.zeros((8 - _C,), jnp.float32)]).reshape(1, 8)

  degp = _deg_kernel()(dst_p, ones8, zero8)
  h1p = _fwd1(x, W1, degp)
  agg1 = _agg_kernel(_H)(h1p, src_p, dst_p, zero16)
  zp = _layer1(agg1, h1p, degp, b1.reshape(1, _H), w2p)
  agg2 = _agg_kernel(8)(zp, src_p, dst_p, zero8)
  out = _layer2(agg2, zp, degp, b2p)
  return out[:, :_C]


# speedup vs baseline: 1.0540x; 1.0170x over previous
import functools

import jax
import jax.numpy as jnp
from jax import lax
from jax.experimental import pallas as pl
from jax.experimental.pallas import tpu as pltpu
from jax.experimental.pallas import tpu_sc as plsc

_N = 10000
_E = 320000
_D = 128
_H = 16
_C = 3

_NC = 2
_NS = 16
_NT = _NC * _NS
_B = 128
_NB = 80
_EP = _NT * _NB * _B
_NPAD = 10112
_RPT = _NPAD // _NS
_NBUF = 8
_PF = 4
_SLAG = 4

_BLK = 5000
_GRID = _N // _BLK


def _deg_body(dst_hbm, ones_hbm, zero_hbm, out_hbm, dst_v, ones_v, acc_sh, sem):
  cid = lax.axis_index("c")
  sid = lax.axis_index("s")
  tile = cid * _NS + sid
  pltpu.sync_copy(dst_hbm.at[tile], dst_v)
  pltpu.sync_copy(ones_hbm, ones_v)
  pltpu.sync_copy(zero_hbm.at[pl.ds(sid * _RPT, _RPT)],
                  acc_sh.at[pl.ds(sid * _RPT, _RPT)])
  plsc.subcore_barrier()

  def body(j, carry):
    @pl.when(j >= _SLAG)
    def _():
      pltpu.make_async_copy(ones_v, acc_sh.at[dst_v.at[j - _SLAG]], sem).wait()

    pltpu.async_copy(ones_v, acc_sh.at[dst_v.at[j]], sem, add=True)
    return carry

  lax.fori_loop(0, _NB, body, 0)

  def drain(j, carry):
    pltpu.make_async_copy(ones_v, acc_sh.at[dst_v.at[j]], sem).wait()
    return carry

  lax.fori_loop(_NB - _SLAG, _NB, drain, 0)
  plsc.subcore_barrier()
  pltpu.sync_copy(acc_sh.at[pl.ds(sid * _RPT, _RPT)],
                  out_hbm.at[cid, pl.ds(sid * _RPT, _RPT)])


@functools.cache
def _deg_kernel():
  return functools.partial(
      pl.kernel,
      out_type=jax.ShapeDtypeStruct((_NC, _NPAD, 8), jnp.float32),
      mesh=plsc.VectorSubcoreMesh(core_axis_name="c", subcore_axis_name="s"),
      scratch_types=[
          pltpu.VMEM((_NB, _B), jnp.int32),
          pltpu.VMEM((_B, 8), jnp.float32),
          pltpu.VMEM_SHARED((_NPAD, 8), jnp.float32),
          pltpu.SemaphoreType.DMA,
      ],
      compiler_params=pltpu.CompilerParams(use_tc_tiling_on_sc=False),
  )(_deg_body)


def _agg_body(hp_hbm, src_hbm, dst_hbm, zero_hbm, out_hbm,
              src_v, dst_v, rows_v, acc_sh, sem_g, sem_s):
  cid = lax.axis_index("c")
  sid = lax.axis_index("s")
  tile = cid * _NS + sid
  pltpu.sync_copy(src_hbm.at[tile], src_v)
  pltpu.sync_copy(dst_hbm.at[tile], dst_v)
  pltpu.sync_copy(zero_hbm.at[pl.ds(sid * _RPT, _RPT)],
                  acc_sh.at[pl.ds(sid * _RPT, _RPT)])
  plsc.subcore_barrier()

  for b in range(_PF):
    pltpu.async_copy(hp_hbm.at[src_v.at[b]], rows_v.at[b], sem_g)

  def body(j, carry):
    b = lax.rem(j, _NBUF)

    @pl.when(j >= _SLAG)
    def _():
      jd = j - _SLAG
      pltpu.make_async_copy(rows_v.at[lax.rem(jd, _NBUF)],
                            acc_sh.at[dst_v.at[jd]], sem_s).wait()

    pltpu.make_async_copy(hp_hbm.at[src_v.at[j]], rows_v.at[b], sem_g).wait()
    pltpu.async_copy(rows_v.at[b], acc_sh.at[dst_v.at[j]], sem_s, add=True)

    @pl.when(j + _PF < _NB)
    def _():
      pltpu.async_copy(hp_hbm.at[src_v.at[j + _PF]],
                       rows_v.at[lax.rem(j + _PF, _NBUF)], sem_g)

    return carry

  lax.fori_loop(0, _NB, body, 0)

  def drain(j, carry):
    pltpu.make_async_copy(rows_v.at[lax.rem(j, _NBUF)],
                          acc_sh.at[dst_v.at[j]], sem_s).wait()
    return carry

  lax.fori_loop(_NB - _SLAG, _NB, drain, 0)
  plsc.subcore_barrier()
  pltpu.sync_copy(acc_sh.at[pl.ds(sid * _RPT, _RPT)],
                  out_hbm.at[cid, pl.ds(sid * _RPT, _RPT)])


@functools.cache
def _agg_kernel(width):
  return functools.partial(
      pl.kernel,
      out_type=jax.ShapeDtypeStruct((_NC, _NPAD, width), jnp.float32),
      mesh=plsc.VectorSubcoreMesh(core_axis_name="c", subcore_axis_name="s"),
      scratch_types=[
          pltpu.VMEM((_NB, _B), jnp.int32),
          pltpu.VMEM((_NB, _B), jnp.int32),
          pltpu.VMEM((_NBUF, _B, width), jnp.float32),
          pltpu.VMEM_SHARED((_NPAD, width), jnp.float32),
          pltpu.SemaphoreType.DMA,
          pltpu.SemaphoreType.DMA,
      ],
      compiler_params=pltpu.CompilerParams(use_tc_tiling_on_sc=False),
  )(_agg_body)


def _dis_of(d_ref):
  deg = 1.0 + d_ref[0, :, :1] + d_ref[1, :, :1]
  return lax.rsqrt(deg)


def _fwd1_body(x_ref, w_ref, d_ref, o_ref):
  h1 = jnp.dot(x_ref[...], w_ref[...], preferred_element_type=jnp.float32)
  o_ref[...] = h1 * _dis_of(d_ref)


def _layer1_body(a_ref, hp_ref, d_ref, b1_ref, w2_ref, o_ref):
  dis = _dis_of(d_ref)
  out1 = dis * (a_ref[0] + a_ref[1] + hp_ref[...]) + b1_ref[...]
  g = dis * jnp.maximum(out1, 0.0)
  o_ref[...] = jnp.dot(g, w2_ref[...], preferred_element_type=jnp.float32)


def _layer2_body(c_ref, zp_ref, d_ref, b2_ref, o_ref):
  out2 = _dis_of(d_ref) * (c_ref[0] + c_ref[1] + zp_ref[...]) + b2_ref[...]
  mask = lax.broadcasted_iota(jnp.int32, (_BLK, 8), 1) < _C
  neg = jnp.float32(-1e30)
  masked = jnp.where(mask, out2, neg)
  m = jnp.max(masked, axis=1, keepdims=True)
  e = jnp.where(mask, jnp.exp(masked - m), 0.0)
  s = jnp.log(jnp.sum(e, axis=1, keepdims=True))
  o_ref[...] = out2 - m - s


def _row_spec(width):
  return pl.BlockSpec((_BLK, width), lambda i: (i, 0))


def _part_spec(width):
  return pl.BlockSpec((_NC, _BLK, width), lambda i: (0, i, 0))


def _full_spec(shape):
  return pl.BlockSpec(shape, lambda i: tuple(0 for _ in shape))


_fwd1 = pl.pallas_call(
    _fwd1_body,
    grid=(_GRID,),
    in_specs=[_row_spec(_D), _full_spec((_D, _H)), _part_spec(8)],
    out_specs=_row_spec(_H),
    out_shape=jax.ShapeDtypeStruct((_N, _H), jnp.float32),
)

_layer1 = pl.pallas_call(
    _layer1_body,
    grid=(_GRID,),
    in_specs=[_part_spec(_H), _row_spec(_H), _part_spec(8),
              _full_spec((1, _H)), _full_spec((_H, 8))],
    out_specs=_row_spec(8),
    out_shape=jax.ShapeDtypeStruct((_N, 8), jnp.float32),
)

_layer2 = pl.pallas_call(
    _layer2_body,
    grid=(_GRID,),
    in_specs=[_part_spec(8), _row_spec(8), _part_spec(8), _full_spec((1, 8))],
    out_specs=_row_spec(8),
    out_shape=jax.ShapeDtypeStruct((_N, 8), jnp.float32),
)


@jax.jit
def kernel(x, edge_index, W1, b1, W2, b2):
  src = edge_index[0]
  dst = edge_index[1]
  pad = _EP - _E
  src_p = jnp.concatenate(
      [src, jnp.zeros((pad,), jnp.int32)]).reshape(_NT, _NB, _B)
  dst_p = jnp.concatenate(
      [dst, jnp.full((pad,), _N, jnp.int32)]).reshape(_NT, _NB, _B)

  ones8 = jnp.ones((_B, 8), jnp.float32)
  zero8 = jnp.zeros((_NPAD, 8), jnp.float32)
  zero16 = jnp.zeros((_NPAD, _H), jnp.float32)
  w2p = jnp.concatenate([W2, jnp.zeros((_H, 8 - _C), jnp.float32)], axis=1)
  b2p = jnp.concatenate([b2, jnp.zeros((8 - _C,), jnp.float32)]).reshape(1, 8)

  degp = _deg_kernel()(dst_p, ones8, zero8)
  h1p = _fwd1(x, W1, degp)
  agg1 = _agg_kernel(_H)(h1p, src_p, dst_p, zero16)
  zp = _layer1(agg1, h1p, degp, b1.reshape(1, _H), w2p)
  agg2 = _agg_kernel(8)(zp, src_p, dst_p, zero8)
  out = _layer2(agg2, zp, degp, b2p)
  return out[:, :_C]
